# baseline (speedup 1.0000x reference)
import jax
import jax.numpy as jnp
from jax import lax
from jax.experimental import pallas as pl
from jax.experimental.pallas import tpu as pltpu
from jax.experimental.pallas import tpu_sc as plsc

_B_TOK = 16384
_SEQ = 50
_D = 32
_B = _B_TOK * _SEQ
_NW = 32
_B_PER_W = _B // _NW
_NBUF = 4
_C = 800
_GROUP = _NBUF * _C
_NG = _B_PER_W // _GROUP


def _emb_body(idx_hbm, table_hbm, out_hbm, idx_v, rows0, rows1, rows2, rows3,
              g0, g1, g2, g3, s0, s1, s2, s3):
    rows = (rows0, rows1, rows2, rows3)
    gsem = (g0, g1, g2, g3)
    ssem = (s0, s1, s2, s3)
    wid = lax.axis_index("s") * 2 + lax.axis_index("c")
    base = wid * _B_PER_W
    pltpu.sync_copy(idx_hbm.at[pl.ds(base, _B_PER_W)], idx_v)
    for b in range(_NBUF):
        pltpu.async_copy(table_hbm.at[idx_v.at[pl.ds(b * _C, _C)]],
                         rows[b], gsem[b])

    def body(g, carry):
        for b in range(_NBUF):
            off = g * _GROUP + b * _C
            pltpu.make_async_copy(table_hbm.at[idx_v.at[pl.ds(0, _C)]],
                                  rows[b], gsem[b]).wait()
            pltpu.async_copy(rows[b], out_hbm.at[pl.ds(base + off, _C)],
                             ssem[b])

        @pl.when(g + 1 < _NG)
        def _():
            for b in range(_NBUF):
                pltpu.make_async_copy(rows[b], out_hbm.at[pl.ds(base, _C)],
                                      ssem[b]).wait()
                noff = (g + 1) * _GROUP + b * _C
                pltpu.async_copy(table_hbm.at[idx_v.at[pl.ds(noff, _C)]],
                                 rows[b], gsem[b])

        return carry

    lax.fori_loop(0, _NG, body, 0)
    for b in range(_NBUF):
        pltpu.make_async_copy(rows[b], out_hbm.at[pl.ds(base, _C)],
                              ssem[b]).wait()


def kernel(x, weight):
    idx = x.reshape(-1).astype(jnp.int32)
    mesh = plsc.VectorSubcoreMesh(core_axis_name="c", subcore_axis_name="s")
    out = pl.kernel(
        _emb_body,
        out_type=jax.ShapeDtypeStruct((_B, _D), jnp.float32),
        mesh=mesh,
        scratch_types=[
            pltpu.VMEM((_B_PER_W,), jnp.int32),
            pltpu.VMEM((_C, _D), jnp.float32),
            pltpu.VMEM((_C, _D), jnp.float32),
            pltpu.VMEM((_C, _D), jnp.float32),
            pltpu.VMEM((_C, _D), jnp.float32),
            pltpu.SemaphoreType.DMA,
            pltpu.SemaphoreType.DMA,
            pltpu.SemaphoreType.DMA,
            pltpu.SemaphoreType.DMA,
            pltpu.SemaphoreType.DMA,
            pltpu.SemaphoreType.DMA,
            pltpu.SemaphoreType.DMA,
            pltpu.SemaphoreType.DMA,
        ],
        compiler_params=pltpu.CompilerParams(use_tc_tiling_on_sc=False),
    )(idx, weight)
    return out.reshape(_B_TOK, _SEQ, _D)


# speedup vs baseline: 1.4631x; 1.4631x over previous
import jax
import jax.numpy as jnp
from jax import lax
from jax.experimental import pallas as pl
from jax.experimental.pallas import tpu as pltpu
from jax.experimental.pallas import tpu_sc as plsc

_NB = 16384
_NS = 50
_D = 32
_NW = 32
_BPW = _NB // _NW
_W = 256
_NPAIR = _NS


def _emb_body(xT, w128, out,
              idx0, idx1, rows0, rows1, offs0, offs1,
              gbuf0, gbuf1, outv0, outv1,
              isem0, isem1, gsem0, gsem1, osem0, osem1):
    idxv = (idx0, idx1)
    rows = (rows0, rows1)
    offs = (offs0, offs1)
    gbuf = (gbuf0, gbuf1)
    outv = (outv0, outv1)
    isem = (isem0, isem1)
    gsem = (gsem0, gsem1)
    osem = (osem0, osem1)

    wid = lax.axis_index("s") * 2 + lax.axis_index("c")
    b0 = wid * _BPW
    iota = lax.iota(jnp.int32, 16)

    def fire_idx(s, h, p):
        pltpu.async_copy(xT.at[s, pl.ds(b0 + h * _W, _W)], idxv[p], isem[p])

    def wait_idx(p):
        pltpu.make_async_copy(xT.at[0, pl.ds(0, _W)], idxv[p], isem[p]).wait()

    def prep(p):
        def body(k, c):
            e = idxv[p][pl.ds(16 * k, 16)]
            rows[p][pl.ds(16 * k, 16)] = e >> 2
            offs[p][pl.ds(16 * k, 16)] = (e & 3) << 5
            return c
        lax.fori_loop(0, _W // 16, body, 0)

    def fire_gather(p):
        pltpu.async_copy(w128.at[rows[p]], gbuf[p], gsem[p])

    def wait_gather(p):
        pltpu.make_async_copy(w128.at[rows[p]], gbuf[p], gsem[p]).wait()

    def extract(p):
        def body(k, c):
            t = iota + 16 * k
            off = offs[p][pl.ds(16 * k, 16)]
            for d in range(_D):
                outv[p][d, pl.ds(16 * k, 16)] = plsc.load_gather(
                    gbuf[p], [t, off + d])
            return c
        lax.fori_loop(0, _W // 16, body, 0)

    def fire_store(s, h, p):
        pltpu.async_copy(outv[p], out.at[s, :, pl.ds(b0 + h * _W, _W)],
                         osem[p])

    def wait_store(p):
        pltpu.make_async_copy(outv[p], out.at[0, :, pl.ds(0, _W)],
                              osem[p]).wait()

    fire_idx(0, 0, 0)
    fire_idx(0, 1, 1)
    wait_idx(0)
    prep(0)
    fire_gather(0)

    def pair(m, carry):
        wait_idx(1)
        prep(1)
        fire_gather(1)

        @pl.when(m + 1 < _NPAIR)
        def _():
            fire_idx(m + 1, 0, 0)

        wait_gather(0)

        @pl.when(m >= 1)
        def _():
            wait_store(0)
        extract(0)
        fire_store(m, 0, 0)

        @pl.when(m + 1 < _NPAIR)
        def _():
            wait_idx(0)
            prep(0)
            fire_gather(0)
            fire_idx(m + 1, 1, 1)

        wait_gather(1)

        @pl.when(m >= 1)
        def _():
            wait_store(1)
        extract(1)
        fire_store(m, 1, 1)
        return carry

    lax.fori_loop(0, _NPAIR, pair, 0)
    wait_store(0)
    wait_store(1)


def kernel(x, weight):
    xT = x.T
    w128 = weight.reshape(250000, 128)
    mesh = plsc.VectorSubcoreMesh(core_axis_name="c", subcore_axis_name="s")
    outT = pl.kernel(
        _emb_body,
        out_type=jax.ShapeDtypeStruct((_NS, _D, _NB), jnp.float32),
        mesh=mesh,
        scratch_types=[
            pltpu.VMEM((_W,), jnp.int32),
            pltpu.VMEM((_W,), jnp.int32),
            pltpu.VMEM((_W,), jnp.int32),
            pltpu.VMEM((_W,), jnp.int32),
            pltpu.VMEM((_W,), jnp.int32),
            pltpu.VMEM((_W,), jnp.int32),
            pltpu.VMEM((_W, 128), jnp.float32),
            pltpu.VMEM((_W, 128), jnp.float32),
            pltpu.VMEM((_D, _W), jnp.float32),
            pltpu.VMEM((_D, _W), jnp.float32),
            pltpu.SemaphoreType.DMA,
            pltpu.SemaphoreType.DMA,
            pltpu.SemaphoreType.DMA,
            pltpu.SemaphoreType.DMA,
            pltpu.SemaphoreType.DMA,
            pltpu.SemaphoreType.DMA,
        ],
        compiler_params=pltpu.CompilerParams(needs_layout_passes=False),
    )(xT, w128)
    return outT.transpose(2, 0, 1)
